# Initial kernel scaffold; baseline (speedup 1.0000x reference)
import functools

import jax
import jax.numpy as jnp
from jax import lax
from jax.experimental import pallas as pl
from jax.experimental.pallas import tpu as pltpu
from jax.experimental.pallas import tpu_sc as plsc

N_NODES = 100000
N_EDGES = 1600000
D_EDGE = 16
D_FEAT = 128
N_OUT = 128

NUM_CORES = 2
NUM_SUBCORES = 16

SUB = 100
K_SUB = 10
BIG = SUB * K_SUB
E_PER_TILE = N_EDGES // NUM_SUBCORES
N_BIG = E_PER_TILE // BIG
ROWS_PER_TILE = N_NODES // NUM_SUBCORES
ZROWS = 625
N_ZCHUNK = ROWS_PER_TILE // ZROWS


def _sc_segment_sums(edge_attr, idx_all):
  mesh = plsc.VectorSubcoreMesh(
      core_axis_name="c", subcore_axis_name="s")

  @functools.partial(
      pl.kernel,
      mesh=mesh,
      out_type=jax.ShapeDtypeStruct((NUM_CORES * N_NODES, D_EDGE),
                                    jnp.float32),
      scratch_types=[
          pltpu.VMEM_SHARED((N_NODES, D_EDGE), jnp.float32),
          pltpu.VMEM((BIG, D_EDGE), jnp.float32),
          pltpu.VMEM((K_SUB, SUB), jnp.int32),
          pltpu.VMEM((ZROWS, D_EDGE), jnp.float32),
      ],
  )
  def seg(edge_hbm, idx_hbm, out_hbm, accum, rows_v, idx_v, zbuf):
    c = lax.axis_index("c")
    s = lax.axis_index("s")

    def zfill(i, carry):
      zbuf[i, :] = jnp.zeros((D_EDGE,), jnp.float32)
      return carry
    lax.fori_loop(0, ZROWS, zfill, 0)
    zbase = s * ROWS_PER_TILE

    def zcopy(t, carry):
      pltpu.sync_copy(zbuf, accum.at[pl.ds(zbase + t * ZROWS, ZROWS)])
      return carry
    lax.fori_loop(0, N_ZCHUNK, zcopy, 0)
    plsc.subcore_barrier()

    ebase = s * E_PER_TILE
    cbase = c * (N_EDGES // SUB) + s * (E_PER_TILE // SUB)

    def window(t, carry):
      pltpu.sync_copy(edge_hbm.at[pl.ds(ebase + t * BIG, BIG)], rows_v)
      pltpu.sync_copy(idx_hbm.at[pl.ds(cbase + t * K_SUB, K_SUB)], idx_v)
      for j in range(K_SUB):
        pltpu.sync_copy(rows_v.at[pl.ds(j * SUB, SUB)],
                        accum.at[idx_v.at[j]], add=True)
      return carry
    lax.fori_loop(0, N_BIG, window, 0)
    plsc.subcore_barrier()

    obase = s * ROWS_PER_TILE
    pltpu.sync_copy(accum.at[pl.ds(obase, ROWS_PER_TILE)],
                    out_hbm.at[pl.ds(c * N_NODES + obase, ROWS_PER_TILE)])

  return seg(edge_attr, idx_all)


def _mm_body(recv_ref, send_ref, node_ref, wr_ref, ws_ref, wn_ref, b_ref,
             out_ref):
  acc = jnp.dot(recv_ref[...], wr_ref[...],
                preferred_element_type=jnp.float32,
                precision=lax.Precision.HIGHEST)
  acc += jnp.dot(send_ref[...], ws_ref[...],
                 preferred_element_type=jnp.float32,
                 precision=lax.Precision.HIGHEST)
  acc += jnp.dot(node_ref[...], wn_ref[...],
                 preferred_element_type=jnp.float32,
                 precision=lax.Precision.HIGHEST)
  acc += b_ref[...]
  out_ref[...] = jnp.maximum(acc, 0.0)


BM = 1000


def _tc_project(recv, send, node_attr, W, b):
  wr = W[0:D_EDGE]
  ws = W[D_EDGE:2 * D_EDGE]
  wn = W[2 * D_EDGE:]
  b2 = b.reshape(1, N_OUT)
  grid = (N_NODES // BM,)
  return pl.pallas_call(
      _mm_body,
      grid=grid,
      in_specs=[
          pl.BlockSpec((BM, D_EDGE), lambda i: (i, 0)),
          pl.BlockSpec((BM, D_EDGE), lambda i: (i, 0)),
          pl.BlockSpec((BM, D_FEAT), lambda i: (i, 0)),
          pl.BlockSpec((D_EDGE, N_OUT), lambda i: (0, 0)),
          pl.BlockSpec((D_EDGE, N_OUT), lambda i: (0, 0)),
          pl.BlockSpec((D_FEAT, N_OUT), lambda i: (0, 0)),
          pl.BlockSpec((1, N_OUT), lambda i: (0, 0)),
      ],
      out_specs=pl.BlockSpec((BM, N_OUT), lambda i: (i, 0)),
      out_shape=jax.ShapeDtypeStruct((N_NODES, N_OUT), jnp.float32),
  )(recv, send, node_attr, wr, ws, wn, b2)


def kernel(node_attr, edge_attr, edge_index, W, b):
  row = edge_index[0].astype(jnp.int32)
  col = edge_index[1].astype(jnp.int32)
  idx_all = jnp.concatenate(
      [col.reshape(N_EDGES // SUB, SUB), row.reshape(N_EDGES // SUB, SUB)],
      axis=0)
  tables = _sc_segment_sums(edge_attr, idx_all)
  recv = tables[:N_NODES]
  send = tables[N_NODES:]
  return _tc_project(recv, send, node_attr, W, b)


# speedup vs baseline: 5.7774x; 5.7774x over previous
import functools

import jax
import jax.numpy as jnp
from jax import lax
from jax.experimental import pallas as pl
from jax.experimental.pallas import tpu as pltpu
from jax.experimental.pallas import tpu_sc as plsc

N_NODES = 100000
N_EDGES = 1600000
D_EDGE = 16
D_FEAT = 128
N_OUT = 128

NUM_CORES = 2
NUM_SUBCORES = 16

SUB = 100
K_SUB = 8
BIG = SUB * K_SUB
E_PER_TILE = N_EDGES // NUM_SUBCORES
N_BIG = E_PER_TILE // BIG
N_PAD = 100352
ROWS_PER_TILE = N_PAD // NUM_SUBCORES
ZROWS = 784
N_ZCHUNK = ROWS_PER_TILE // ZROWS


def _sc_segment_sums(edge_attr, idx_all):
  mesh = plsc.VectorSubcoreMesh(
      core_axis_name="c", subcore_axis_name="s")

  @functools.partial(
      pl.kernel,
      mesh=mesh,
      out_type=jax.ShapeDtypeStruct((NUM_CORES * N_PAD, D_EDGE),
                                    jnp.float32),
      scratch_types=[
          pltpu.VMEM_SHARED((N_PAD, D_EDGE), jnp.float32),
          pltpu.VMEM((BIG, D_EDGE), jnp.float32),
          pltpu.VMEM((K_SUB, SUB), jnp.int32),
          pltpu.VMEM((ZROWS, D_EDGE), jnp.float32),
      ],
      compiler_params=pltpu.CompilerParams(use_tc_tiling_on_sc=False),
  )
  def seg(edge_hbm, idx_hbm, out_hbm, accum, rows_v, idx_v, zbuf):
    c = lax.axis_index("c")
    s = lax.axis_index("s")

    def zfill(i, carry):
      zbuf[i, :] = jnp.zeros((D_EDGE,), jnp.float32)
      return carry
    lax.fori_loop(0, ZROWS, zfill, 0)
    zbase = s * ROWS_PER_TILE

    def zcopy(t, carry):
      pltpu.sync_copy(zbuf, accum.at[pl.ds(zbase + t * ZROWS, ZROWS)])
      return carry
    lax.fori_loop(0, N_ZCHUNK, zcopy, 0)
    plsc.subcore_barrier()

    ebase = s * E_PER_TILE
    cbase = c * (N_EDGES // SUB) + s * (E_PER_TILE // SUB)

    def window(t, carry):
      pltpu.sync_copy(edge_hbm.at[pl.ds(ebase + t * BIG, BIG)], rows_v)
      pltpu.sync_copy(idx_hbm.at[pl.ds(cbase + t * K_SUB, K_SUB)], idx_v)
      for j in range(K_SUB):
        pltpu.sync_copy(rows_v.at[pl.ds(j * SUB, SUB)],
                        accum.at[idx_v.at[j]], add=True)
      return carry
    lax.fori_loop(0, N_BIG, window, 0)
    plsc.subcore_barrier()

    obase = s * ROWS_PER_TILE
    pltpu.sync_copy(accum.at[pl.ds(obase, ROWS_PER_TILE)],
                    out_hbm.at[pl.ds(c * N_PAD + obase, ROWS_PER_TILE)])

  return seg(edge_attr, idx_all)


def _mm_body(recv_ref, send_ref, node_ref, wr_ref, ws_ref, wn_ref, b_ref,
             out_ref):
  acc = jnp.dot(recv_ref[...], wr_ref[...],
                preferred_element_type=jnp.float32,
                precision=lax.Precision.HIGHEST)
  acc += jnp.dot(send_ref[...], ws_ref[...],
                 preferred_element_type=jnp.float32,
                 precision=lax.Precision.HIGHEST)
  acc += jnp.dot(node_ref[...], wn_ref[...],
                 preferred_element_type=jnp.float32,
                 precision=lax.Precision.HIGHEST)
  acc += b_ref[...]
  out_ref[...] = jnp.maximum(acc, 0.0)


BM = 1000


def _tc_project(recv, send, node_attr, W, b):
  wr = W[0:D_EDGE]
  ws = W[D_EDGE:2 * D_EDGE]
  wn = W[2 * D_EDGE:]
  b2 = b.reshape(1, N_OUT)
  grid = (N_NODES // BM,)
  return pl.pallas_call(
      _mm_body,
      grid=grid,
      in_specs=[
          pl.BlockSpec((BM, D_EDGE), lambda i: (i, 0)),
          pl.BlockSpec((BM, D_EDGE), lambda i: (i, 0)),
          pl.BlockSpec((BM, D_FEAT), lambda i: (i, 0)),
          pl.BlockSpec((D_EDGE, N_OUT), lambda i: (0, 0)),
          pl.BlockSpec((D_EDGE, N_OUT), lambda i: (0, 0)),
          pl.BlockSpec((D_FEAT, N_OUT), lambda i: (0, 0)),
          pl.BlockSpec((1, N_OUT), lambda i: (0, 0)),
      ],
      out_specs=pl.BlockSpec((BM, N_OUT), lambda i: (i, 0)),
      out_shape=jax.ShapeDtypeStruct((N_NODES, N_OUT), jnp.float32),
  )(recv, send, node_attr, wr, ws, wn, b2)


def kernel(node_attr, edge_attr, edge_index, W, b):
  row = edge_index[0].astype(jnp.int32)
  col = edge_index[1].astype(jnp.int32)
  idx_all = jnp.concatenate(
      [col.reshape(N_EDGES // SUB, SUB), row.reshape(N_EDGES // SUB, SUB)],
      axis=0)
  tables = _sc_segment_sums(edge_attr, idx_all)
  recv = tables[:N_NODES]
  send = tables[N_PAD:N_PAD + N_NODES]
  return _tc_project(recv, send, node_attr, W, b)
